# baseline (speedup 1.0000x reference)
import functools
import jax
import jax.numpy as jnp
from jax import lax
from jax.experimental import pallas as pl
from jax.experimental.pallas import tpu as pltpu
from jax.experimental.pallas import tpu_sc as plsc

_K = 8
_EPS = 1e-12
_BT = 8


def _bf(x):
    return x.astype(jnp.bfloat16)


def _dot(a, b):
    return jnp.dot(a, b, preferred_element_type=jnp.float32)


def _fwd_kernel(xb_ref, w1tb_ref, w2tb_ref, w1t_ref, b1_ref, b2_ref,
                z1_ref, a2_ref, n1_ref):
    xb = xb_ref[...]
    z1 = _dot(xb, w1tb_ref[...]) + b1_ref[...]
    z1_ref[...] = z1[:, None, :]
    a1 = jnp.where(z1 > 0.0, b1_ref[...], 0.0)
    a2 = _dot(_bf(a1), w2tb_ref[...]) + b2_ref[...]
    a2_ref[...] = a2[:, None, :]
    w1t = w1t_ref[...]
    n1_ref[...] = jnp.sqrt(jnp.sum(w1t * w1t, axis=0, keepdims=True))


def _dist_kernel(xb_ref, w1tb_ref, w2tb_ref, w3tb_ref, z1_ref, a2_ref,
                 n1_ref, b3_ref, out_ref):
    bt = z1_ref.shape[0]
    d = w1tb_ref.shape[0]
    w1tb = w1tb_ref[...]
    bf0 = jnp.bfloat16(0)
    z1_rows = [z1_ref[i] for i in range(bt)]
    a1tb = jnp.concatenate(
        [jnp.where(z1_rows[i] > 0.0, w1tb, bf0) for i in range(bt)],
        axis=0)
    v2t_all = _dot(a1tb, w2tb_ref[...])
    v2tb_all = _bf(v2t_all)
    n2_rows, z2_rows, m2_rows = [], [], []
    for i in range(bt):
        v2t_i = v2t_all[i * d:(i + 1) * d]
        n2_rows.append(jnp.sqrt(jnp.sum(v2t_i * v2t_i, axis=0, keepdims=True)))
        z2 = _dot(xb_ref[i], v2tb_all[i * d:(i + 1) * d]) + a2_ref[i]
        z2_rows.append(z2)
        m2_rows.append(z2 > 0.0)
    v2mtb = jnp.concatenate(
        [jnp.where(m2_rows[i], v2tb_all[i * d:(i + 1) * d], bf0)
         for i in range(bt)], axis=0)
    v3t_all = _dot(v2mtb, w3tb_ref[...])
    v3tb_all = _bf(v3t_all)
    dist_rows = []
    for i in range(bt):
        v3t_i = v3t_all[i * d:(i + 1) * d]
        n3 = jnp.sqrt(jnp.sum(v3t_i * v3t_i, axis=0, keepdims=True))
        a2m = jnp.where(m2_rows[i], a2_ref[i], 0.0)
        a3 = _dot(_bf(a2m), w3tb_ref[...]) + b3_ref[...]
        z3 = _dot(xb_ref[i], v3tb_all[i * d:(i + 1) * d]) + a3
        d1 = jnp.abs(z1_rows[i]) / (n1_ref[...] + _EPS)
        d2 = jnp.abs(z2_rows[i]) / (n2_rows[i] + _EPS)
        d3 = jnp.abs(z3) / (n3 + _EPS)
        dist_rows.append(jnp.concatenate([d1, d2, d3], axis=1))
    out_ref[...] = jnp.concatenate(dist_rows, axis=0)[:, None, :]


def _sc_select(d_hbm, out_hbm, rows_v, acc_v):
    nc = 2
    wid = lax.axis_index("s") * nc + lax.axis_index("c")
    rows = rows_v.shape[0]
    n = rows_v.shape[1]
    pltpu.sync_copy(d_hbm.at[pl.ds(wid * rows, rows)], rows_v)
    lanes = lax.iota(jnp.int32, 16)
    inf16 = jnp.full((16,), jnp.inf, dtype=jnp.float32)
    zero16 = jnp.zeros((16,), dtype=jnp.float32)
    one16i = jnp.ones((16,), dtype=jnp.int32)
    k16i = jnp.full((16,), _K, dtype=jnp.int32)
    acc = zero16
    for r in range(rows):
        def chunk_body(c, keep):
            t = rows_v[r, pl.ds(c * 16, 16)]
            out = []
            for j in range(_K):
                out.append(jnp.minimum(keep[j], t))
                t = jnp.maximum(keep[j], t)
            return tuple(out)
        keep = list(lax.fori_loop(0, n // 16, chunk_body, (inf16,) * _K))
        need = k16i
        total = zero16
        for _ in range(_K):
            m = keep[0]
            for j in range(1, _K):
                m = jnp.minimum(m, keep[j])
            for sh in (8, 4, 2, 1):
                acc_v[...] = m
                m = jnp.minimum(m, plsc.load_gather(acc_v, [(lanes + sh) & 15]))
            cnt = jnp.zeros((16,), dtype=jnp.int32)
            eqs = []
            for j in range(_K):
                eq = keep[j] == m
                eqs.append(eq)
                cnt = cnt + plsc.all_reduce_population_count(eq)
            take = jnp.minimum(cnt, need)
            need = need - take
            contrib = m * take.astype(jnp.float32)
            total = total + jnp.where(take > 0, contrib, zero16)
            for j in range(_K):
                keep[j] = jnp.where(eqs[j], inf16, keep[j])
        acc = jnp.where(lanes == r, total, acc)
    acc_v[...] = acc
    pltpu.sync_copy(acc_v, out_hbm.at[wid])


def _select_kernel(d_ref, out_ref):
    v = d_ref[...].reshape(d_ref.shape[0], d_ref.shape[2])
    b, n = v.shape
    idx = jax.lax.broadcasted_iota(jnp.int32, (b, n), 1)
    acc = jnp.zeros((), dtype=jnp.float32)
    for _ in range(_K):
        row_min = jnp.min(v, axis=1, keepdims=True)
        acc = acc + jnp.sum(row_min)
        is_min = v == row_min
        min_idx = jnp.min(jnp.where(is_min, idx, n), axis=1, keepdims=True)
        v = jnp.where(idx == min_idx, jnp.float32(jnp.inf), v)
    out_ref[...] = jnp.broadcast_to(acc / _K, (1, 1))


@jax.jit
def kernel(inputs, W1, b1, W2, b2, W3, b3):
    B, d_in = inputs.shape
    h1 = W1.shape[0]
    h2 = W2.shape[0]
    h3 = W3.shape[0]
    w1t = W1.T
    w1tb = w1t.astype(jnp.bfloat16)
    w2tb = W2.T.astype(jnp.bfloat16)
    w3tb = W3.T.astype(jnp.bfloat16)
    xb = inputs.astype(jnp.bfloat16)
    b1r = b1[None, :]
    b2r = b2[None, :]
    b3r = b3[None, :]

    z1, a2, n1 = pl.pallas_call(
        _fwd_kernel,
        out_shape=[
            jax.ShapeDtypeStruct((B, 1, h1), jnp.float32),
            jax.ShapeDtypeStruct((B, 1, h2), jnp.float32),
            jax.ShapeDtypeStruct((1, h1), jnp.float32),
        ],
    )(xb, w1tb, w2tb, w1t, b1r, b2r)

    n_total = h1 + h2 + h3
    xb3 = xb[:, None, :]
    bt = _BT
    n_workers = 32

    def dist_call(xb3_h, z1_h, a2_h):
        bh = xb3_h.shape[0]
        return pl.pallas_call(
            _dist_kernel,
            grid=(bh // bt,),
            in_specs=[
                pl.BlockSpec((bt, 1, d_in), lambda b: (b, 0, 0)),
                pl.BlockSpec((d_in, h1), lambda b: (0, 0)),
                pl.BlockSpec((h1, h2), lambda b: (0, 0)),
                pl.BlockSpec((h2, h3), lambda b: (0, 0)),
                pl.BlockSpec((bt, 1, h1), lambda b: (b, 0, 0)),
                pl.BlockSpec((bt, 1, h2), lambda b: (b, 0, 0)),
                pl.BlockSpec((1, h1), lambda b: (0, 0)),
                pl.BlockSpec((1, h3), lambda b: (0, 0)),
            ],
            out_specs=pl.BlockSpec((bt, 1, n_total), lambda b: (b, 0, 0)),
            out_shape=jax.ShapeDtypeStruct((bh, 1, n_total), jnp.float32),
        )(xb3_h, w1tb, w2tb, w3tb, z1_h, a2_h, n1, b3r)

    def select_call(dists_h):
        bh = dists_h.shape[0]
        sc_fn = functools.partial(
            pl.kernel,
            mesh=plsc.VectorSubcoreMesh(core_axis_name="c",
                                        subcore_axis_name="s"),
            out_type=jax.ShapeDtypeStruct((n_workers, 16), jnp.float32),
            compiler_params=pltpu.CompilerParams(needs_layout_passes=False),
            scratch_types=[
                pltpu.VMEM((bh // n_workers, n_total), jnp.float32),
                pltpu.VMEM((16,), jnp.float32),
            ],
        )(_sc_select)
        return sc_fn(dists_h.reshape(bh, n_total))

    hb = B // 2
    dists0 = dist_call(xb3[:hb], z1[:hb], a2[:hb])
    part0 = select_call(dists0)
    dists1 = dist_call(xb3[hb:], z1[hb:], a2[hb:])
    part1 = select_call(dists1)

    border_dist_sum = (jnp.sum(part0) + jnp.sum(part1)) / _K
    fct_dist_sum = jnp.zeros((), dtype=inputs.dtype)
    return (border_dist_sum, fct_dist_sum)


# speedup vs baseline: 1.0332x; 1.0332x over previous
import functools
import jax
import jax.numpy as jnp
from jax import lax
from jax.experimental import pallas as pl
from jax.experimental.pallas import tpu as pltpu
from jax.experimental.pallas import tpu_sc as plsc

_K = 8
_EPS = 1e-12
_BT = 8


def _bf(x):
    return x.astype(jnp.bfloat16)


def _dot(a, b):
    return jnp.dot(a, b, preferred_element_type=jnp.float32)


def _fwd_kernel(xb_ref, w1tb_ref, w2tb_ref, w1t_ref, b1_ref, b2_ref,
                z1_ref, a2_ref, n1_ref):
    xb = xb_ref[...]
    z1 = _dot(xb, w1tb_ref[...]) + b1_ref[...]
    z1_ref[...] = z1[:, None, :]
    a1 = jnp.where(z1 > 0.0, b1_ref[...], 0.0)
    a2 = _dot(_bf(a1), w2tb_ref[...]) + b2_ref[...]
    a2_ref[...] = a2[:, None, :]
    w1t = w1t_ref[...]
    n1_ref[...] = jnp.sqrt(jnp.sum(w1t * w1t, axis=0, keepdims=True))


def _dist_kernel(xb_ref, w1tb_ref, w2tb_ref, w3tb_ref, z1_ref, a2_ref,
                 n1_ref, b3_ref, out_ref):
    bt = z1_ref.shape[0]
    d = w1tb_ref.shape[0]
    w1tb = w1tb_ref[...]
    bf0 = jnp.bfloat16(0)
    z1_rows = [z1_ref[i] for i in range(bt)]
    a1tb = jnp.concatenate(
        [jnp.where(z1_rows[i] > 0.0, w1tb, bf0) for i in range(bt)],
        axis=0)
    v2t_all = _dot(a1tb, w2tb_ref[...])
    v2tb_all = _bf(v2t_all)
    n2_rows, z2_rows, m2_rows = [], [], []
    for i in range(bt):
        v2t_i = v2t_all[i * d:(i + 1) * d]
        n2_rows.append(jnp.sqrt(jnp.sum(v2t_i * v2t_i, axis=0, keepdims=True)))
        z2 = _dot(xb_ref[i], v2tb_all[i * d:(i + 1) * d]) + a2_ref[i]
        z2_rows.append(z2)
        m2_rows.append(z2 > 0.0)
    v2mtb = jnp.concatenate(
        [jnp.where(m2_rows[i], v2tb_all[i * d:(i + 1) * d], bf0)
         for i in range(bt)], axis=0)
    v3t_all = _dot(v2mtb, w3tb_ref[...])
    v3tb_all = _bf(v3t_all)
    dist_rows = []
    for i in range(bt):
        v3t_i = v3t_all[i * d:(i + 1) * d]
        n3 = jnp.sqrt(jnp.sum(v3t_i * v3t_i, axis=0, keepdims=True))
        a2m = jnp.where(m2_rows[i], a2_ref[i], 0.0)
        a3 = _dot(_bf(a2m), w3tb_ref[...]) + b3_ref[...]
        z3 = _dot(xb_ref[i], v3tb_all[i * d:(i + 1) * d]) + a3
        d1 = jnp.abs(z1_rows[i]) / (n1_ref[...] + _EPS)
        d2 = jnp.abs(z2_rows[i]) / (n2_rows[i] + _EPS)
        d3 = jnp.abs(z3) / (n3 + _EPS)
        dist_rows.append(jnp.concatenate([d1, d2, d3], axis=1))
    out_ref[...] = jnp.concatenate(dist_rows, axis=0)[:, None, :]


def _sc_select(d_hbm, out_hbm, rows_v, stage_v):
    nc = 2
    wid = lax.axis_index("s") * nc + lax.axis_index("c")
    rows = rows_v.shape[0]
    n = rows_v.shape[1]
    pltpu.sync_copy(d_hbm.at[pl.ds(wid * rows, rows)], rows_v)
    lanes = lax.iota(jnp.int32, 16)
    inf16 = jnp.full((16,), jnp.inf, dtype=jnp.float32)
    zero16 = jnp.zeros((16,), dtype=jnp.float32)
    k16i = jnp.full((16,), _K, dtype=jnp.int32)

    def chunk_body(c, carry):
        new = []
        for r in range(rows):
            keep = list(carry[r])
            t = rows_v[r, pl.ds(c * 16, 16)]
            for j in range(_K):
                lo = jnp.minimum(keep[j], t)
                t = jnp.maximum(keep[j], t)
                keep[j] = lo
            new.append(tuple(keep))
        return tuple(new)

    init = tuple((inf16,) * _K for _ in range(rows))
    keeps = [list(ks) for ks in lax.fori_loop(0, n // 16, chunk_body, init)]

    need = [k16i] * rows
    total = [zero16] * rows
    for _ in range(_K):
        ms = []
        for r in range(rows):
            m = keeps[r][0]
            for j in range(1, _K):
                m = jnp.minimum(m, keeps[r][j])
            ms.append(m)
        for sh in (8, 4, 2, 1):
            for r in range(rows):
                stage_v[r] = ms[r]
            for r in range(rows):
                g = plsc.load_gather(
                    stage_v,
                    [jnp.full((16,), r, jnp.int32), (lanes + sh) & 15])
                ms[r] = jnp.minimum(ms[r], g)
        for r in range(rows):
            cnt = jnp.zeros((16,), dtype=jnp.int32)
            eqs = []
            for j in range(_K):
                eq = keeps[r][j] == ms[r]
                eqs.append(eq)
                cnt = cnt + plsc.all_reduce_population_count(eq)
            take = jnp.minimum(cnt, need[r])
            need[r] = need[r] - take
            contrib = ms[r] * take.astype(jnp.float32)
            total[r] = total[r] + jnp.where(take > 0, contrib, zero16)
            for j in range(_K):
                keeps[r][j] = jnp.where(eqs[j], inf16, keeps[r][j])
    acc = zero16
    for r in range(rows):
        acc = jnp.where(lanes == r, total[r], acc)
    stage_v[0] = acc
    pltpu.sync_copy(stage_v.at[0], out_hbm.at[wid])


def _select_kernel(d_ref, out_ref):
    v = d_ref[...].reshape(d_ref.shape[0], d_ref.shape[2])
    b, n = v.shape
    idx = jax.lax.broadcasted_iota(jnp.int32, (b, n), 1)
    acc = jnp.zeros((), dtype=jnp.float32)
    for _ in range(_K):
        row_min = jnp.min(v, axis=1, keepdims=True)
        acc = acc + jnp.sum(row_min)
        is_min = v == row_min
        min_idx = jnp.min(jnp.where(is_min, idx, n), axis=1, keepdims=True)
        v = jnp.where(idx == min_idx, jnp.float32(jnp.inf), v)
    out_ref[...] = jnp.broadcast_to(acc / _K, (1, 1))


@jax.jit
def kernel(inputs, W1, b1, W2, b2, W3, b3):
    B, d_in = inputs.shape
    h1 = W1.shape[0]
    h2 = W2.shape[0]
    h3 = W3.shape[0]
    w1t = W1.T
    w1tb = w1t.astype(jnp.bfloat16)
    w2tb = W2.T.astype(jnp.bfloat16)
    w3tb = W3.T.astype(jnp.bfloat16)
    xb = inputs.astype(jnp.bfloat16)
    b1r = b1[None, :]
    b2r = b2[None, :]
    b3r = b3[None, :]

    z1, a2, n1 = pl.pallas_call(
        _fwd_kernel,
        out_shape=[
            jax.ShapeDtypeStruct((B, 1, h1), jnp.float32),
            jax.ShapeDtypeStruct((B, 1, h2), jnp.float32),
            jax.ShapeDtypeStruct((1, h1), jnp.float32),
        ],
    )(xb, w1tb, w2tb, w1t, b1r, b2r)

    n_total = h1 + h2 + h3
    xb3 = xb[:, None, :]
    bt = _BT
    n_workers = 32

    def dist_call(xb3_h, z1_h, a2_h):
        bh = xb3_h.shape[0]
        return pl.pallas_call(
            _dist_kernel,
            grid=(bh // bt,),
            in_specs=[
                pl.BlockSpec((bt, 1, d_in), lambda b: (b, 0, 0)),
                pl.BlockSpec((d_in, h1), lambda b: (0, 0)),
                pl.BlockSpec((h1, h2), lambda b: (0, 0)),
                pl.BlockSpec((h2, h3), lambda b: (0, 0)),
                pl.BlockSpec((bt, 1, h1), lambda b: (b, 0, 0)),
                pl.BlockSpec((bt, 1, h2), lambda b: (b, 0, 0)),
                pl.BlockSpec((1, h1), lambda b: (0, 0)),
                pl.BlockSpec((1, h3), lambda b: (0, 0)),
            ],
            out_specs=pl.BlockSpec((bt, 1, n_total), lambda b: (b, 0, 0)),
            out_shape=jax.ShapeDtypeStruct((bh, 1, n_total), jnp.float32),
        )(xb3_h, w1tb, w2tb, w3tb, z1_h, a2_h, n1, b3r)

    def select_call(dists_h):
        bh = dists_h.shape[0]
        sc_fn = functools.partial(
            pl.kernel,
            mesh=plsc.VectorSubcoreMesh(core_axis_name="c",
                                        subcore_axis_name="s"),
            out_type=jax.ShapeDtypeStruct((n_workers, 16), jnp.float32),
            compiler_params=pltpu.CompilerParams(needs_layout_passes=False),
            scratch_types=[
                pltpu.VMEM((bh // n_workers, n_total), jnp.float32),
                pltpu.VMEM((bh // n_workers, 16), jnp.float32),
            ],
        )(_sc_select)
        return sc_fn(dists_h.reshape(bh, n_total))

    dists = dist_call(xb3, z1, a2)
    part = select_call(dists)

    border_dist_sum = jnp.sum(part) / _K
    fct_dist_sum = jnp.zeros((), dtype=inputs.dtype)
    return (border_dist_sum, fct_dist_sum)


# speedup vs baseline: 1.0461x; 1.0125x over previous
import functools
import jax
import jax.numpy as jnp
from jax import lax
from jax.experimental import pallas as pl
from jax.experimental.pallas import tpu as pltpu
from jax.experimental.pallas import tpu_sc as plsc

_K = 8
_EPS = 1e-12
_BT = 16


def _bf(x):
    return x.astype(jnp.bfloat16)


def _dot(a, b):
    return jnp.dot(a, b, preferred_element_type=jnp.float32)


def _fwd_kernel(xb_ref, w1tb_ref, w2tb_ref, w1t_ref, b1_ref, b2_ref,
                z1_ref, a2_ref, n1_ref):
    xb = xb_ref[...]
    z1 = _dot(xb, w1tb_ref[...]) + b1_ref[...]
    z1_ref[...] = z1[:, None, :]
    a1 = jnp.where(z1 > 0.0, b1_ref[...], 0.0)
    a2 = _dot(_bf(a1), w2tb_ref[...]) + b2_ref[...]
    a2_ref[...] = a2[:, None, :]
    w1t = w1t_ref[...]
    n1_ref[...] = jnp.sqrt(jnp.sum(w1t * w1t, axis=0, keepdims=True))


def _dist_kernel(xb_ref, w1tb_ref, w2tb_ref, w3tb_ref, z1_ref, a2_ref,
                 n1_ref, b3_ref, out_ref):
    bt = z1_ref.shape[0]
    d = w1tb_ref.shape[0]
    w1tb = w1tb_ref[...]
    bf0 = jnp.bfloat16(0)
    z1_rows = [z1_ref[i] for i in range(bt)]
    a1tb = jnp.concatenate(
        [jnp.where(z1_rows[i] > 0.0, w1tb, bf0) for i in range(bt)],
        axis=0)
    v2t_all = _dot(a1tb, w2tb_ref[...])
    v2tb_all = _bf(v2t_all)
    n2_rows, z2_rows, m2_rows = [], [], []
    for i in range(bt):
        v2t_i = v2t_all[i * d:(i + 1) * d]
        n2_rows.append(jnp.sqrt(jnp.sum(v2t_i * v2t_i, axis=0, keepdims=True)))
        z2 = _dot(xb_ref[i], v2tb_all[i * d:(i + 1) * d]) + a2_ref[i]
        z2_rows.append(z2)
        m2_rows.append(z2 > 0.0)
    v2mtb = jnp.concatenate(
        [jnp.where(m2_rows[i], v2tb_all[i * d:(i + 1) * d], bf0)
         for i in range(bt)], axis=0)
    v3t_all = _dot(v2mtb, w3tb_ref[...])
    v3tb_all = _bf(v3t_all)
    dist_rows = []
    for i in range(bt):
        v3t_i = v3t_all[i * d:(i + 1) * d]
        n3 = jnp.sqrt(jnp.sum(v3t_i * v3t_i, axis=0, keepdims=True))
        a2m = jnp.where(m2_rows[i], a2_ref[i], 0.0)
        a3 = _dot(_bf(a2m), w3tb_ref[...]) + b3_ref[...]
        z3 = _dot(xb_ref[i], v3tb_all[i * d:(i + 1) * d]) + a3
        d1 = jnp.abs(z1_rows[i]) / (n1_ref[...] + _EPS)
        d2 = jnp.abs(z2_rows[i]) / (n2_rows[i] + _EPS)
        d3 = jnp.abs(z3) / (n3 + _EPS)
        dist_rows.append(jnp.concatenate([d1, d2, d3], axis=1))
    out_ref[...] = jnp.concatenate(dist_rows, axis=0)[:, None, :]


def _sc_select(d_hbm, out_hbm, rows_v, stage_v):
    nc = 2
    wid = lax.axis_index("s") * nc + lax.axis_index("c")
    rows = rows_v.shape[0]
    n = rows_v.shape[1]
    pltpu.sync_copy(d_hbm.at[pl.ds(wid * rows, rows)], rows_v)
    lanes = lax.iota(jnp.int32, 16)
    inf16 = jnp.full((16,), jnp.inf, dtype=jnp.float32)
    zero16 = jnp.zeros((16,), dtype=jnp.float32)
    k16i = jnp.full((16,), _K, dtype=jnp.int32)

    def chunk_body(c, carry):
        new = []
        for r in range(rows):
            keep = list(carry[r])
            t = rows_v[r, pl.ds(c * 16, 16)]
            for j in range(_K):
                lo = jnp.minimum(keep[j], t)
                t = jnp.maximum(keep[j], t)
                keep[j] = lo
            new.append(tuple(keep))
        return tuple(new)

    init = tuple((inf16,) * _K for _ in range(rows))
    keeps = [list(ks) for ks in lax.fori_loop(0, n // 16, chunk_body, init)]

    need = [k16i] * rows
    total = [zero16] * rows
    for _ in range(_K):
        ms = []
        for r in range(rows):
            m = keeps[r][0]
            for j in range(1, _K):
                m = jnp.minimum(m, keeps[r][j])
            ms.append(m)
        for sh in (8, 4, 2, 1):
            for r in range(rows):
                stage_v[r] = ms[r]
            for r in range(rows):
                g = plsc.load_gather(
                    stage_v,
                    [jnp.full((16,), r, jnp.int32), (lanes + sh) & 15])
                ms[r] = jnp.minimum(ms[r], g)
        for r in range(rows):
            cnt = jnp.zeros((16,), dtype=jnp.int32)
            eqs = []
            for j in range(_K):
                eq = keeps[r][j] == ms[r]
                eqs.append(eq)
                cnt = cnt + plsc.all_reduce_population_count(eq)
            take = jnp.minimum(cnt, need[r])
            need[r] = need[r] - take
            contrib = ms[r] * take.astype(jnp.float32)
            total[r] = total[r] + jnp.where(take > 0, contrib, zero16)
            for j in range(_K):
                keeps[r][j] = jnp.where(eqs[j], inf16, keeps[r][j])
    acc = zero16
    for r in range(rows):
        acc = jnp.where(lanes == r, total[r], acc)
    stage_v[0] = acc
    pltpu.sync_copy(stage_v.at[0], out_hbm.at[wid])


def _select_kernel(d_ref, out_ref):
    v = d_ref[...].reshape(d_ref.shape[0], d_ref.shape[2])
    b, n = v.shape
    idx = jax.lax.broadcasted_iota(jnp.int32, (b, n), 1)
    acc = jnp.zeros((), dtype=jnp.float32)
    for _ in range(_K):
        row_min = jnp.min(v, axis=1, keepdims=True)
        acc = acc + jnp.sum(row_min)
        is_min = v == row_min
        min_idx = jnp.min(jnp.where(is_min, idx, n), axis=1, keepdims=True)
        v = jnp.where(idx == min_idx, jnp.float32(jnp.inf), v)
    out_ref[...] = jnp.broadcast_to(acc / _K, (1, 1))


@jax.jit
def kernel(inputs, W1, b1, W2, b2, W3, b3):
    B, d_in = inputs.shape
    h1 = W1.shape[0]
    h2 = W2.shape[0]
    h3 = W3.shape[0]
    w1t = W1.T
    w1tb = w1t.astype(jnp.bfloat16)
    w2tb = W2.T.astype(jnp.bfloat16)
    w3tb = W3.T.astype(jnp.bfloat16)
    xb = inputs.astype(jnp.bfloat16)
    b1r = b1[None, :]
    b2r = b2[None, :]
    b3r = b3[None, :]

    z1, a2, n1 = pl.pallas_call(
        _fwd_kernel,
        out_shape=[
            jax.ShapeDtypeStruct((B, 1, h1), jnp.float32),
            jax.ShapeDtypeStruct((B, 1, h2), jnp.float32),
            jax.ShapeDtypeStruct((1, h1), jnp.float32),
        ],
    )(xb, w1tb, w2tb, w1t, b1r, b2r)

    n_total = h1 + h2 + h3
    xb3 = xb[:, None, :]
    bt = _BT
    n_workers = 32

    def dist_call(xb3_h, z1_h, a2_h):
        bh = xb3_h.shape[0]
        return pl.pallas_call(
            _dist_kernel,
            grid=(bh // bt,),
            in_specs=[
                pl.BlockSpec((bt, 1, d_in), lambda b: (b, 0, 0)),
                pl.BlockSpec((d_in, h1), lambda b: (0, 0)),
                pl.BlockSpec((h1, h2), lambda b: (0, 0)),
                pl.BlockSpec((h2, h3), lambda b: (0, 0)),
                pl.BlockSpec((bt, 1, h1), lambda b: (b, 0, 0)),
                pl.BlockSpec((bt, 1, h2), lambda b: (b, 0, 0)),
                pl.BlockSpec((1, h1), lambda b: (0, 0)),
                pl.BlockSpec((1, h3), lambda b: (0, 0)),
            ],
            out_specs=pl.BlockSpec((bt, 1, n_total), lambda b: (b, 0, 0)),
            out_shape=jax.ShapeDtypeStruct((bh, 1, n_total), jnp.float32),
        )(xb3_h, w1tb, w2tb, w3tb, z1_h, a2_h, n1, b3r)

    def select_call(dists_h):
        bh = dists_h.shape[0]
        sc_fn = functools.partial(
            pl.kernel,
            mesh=plsc.VectorSubcoreMesh(core_axis_name="c",
                                        subcore_axis_name="s"),
            out_type=jax.ShapeDtypeStruct((n_workers, 16), jnp.float32),
            compiler_params=pltpu.CompilerParams(needs_layout_passes=False),
            scratch_types=[
                pltpu.VMEM((bh // n_workers, n_total), jnp.float32),
                pltpu.VMEM((bh // n_workers, 16), jnp.float32),
            ],
        )(_sc_select)
        return sc_fn(dists_h.reshape(bh, n_total))

    dists = dist_call(xb3, z1, a2)
    part = select_call(dists)

    border_dist_sum = jnp.sum(part) / _K
    fct_dist_sum = jnp.zeros((), dtype=inputs.dtype)
    return (border_dist_sum, fct_dist_sum)


# speedup vs baseline: 1.1039x; 1.0552x over previous
import functools
import jax
import jax.numpy as jnp
from jax import lax
from jax.experimental import pallas as pl
from jax.experimental.pallas import tpu as pltpu
from jax.experimental.pallas import tpu_sc as plsc

_K = 8
_EPS = 1e-12
_BT = 16


def _bf(x):
    return x.astype(jnp.bfloat16)


def _dot(a, b):
    return jnp.dot(a, b, preferred_element_type=jnp.float32)


def _fwd_kernel(x_ref, w1_ref, w2_ref, w3_ref, b1_ref, b2_ref,
                z1_ref, a2_ref, n1_ref, w1tb_ref, w2tb_ref, w3tb_ref,
                xb3_ref):
    w1t = w1_ref[...].T
    w1tb = _bf(w1t)
    w1tb_ref[...] = w1tb
    w2tb = _bf(w2_ref[...].T)
    w2tb_ref[...] = w2tb
    w3tb_ref[...] = _bf(w3_ref[...].T)
    xb = _bf(x_ref[...])
    xb3_ref[...] = xb[:, None, :]
    z1 = _dot(xb, w1tb) + b1_ref[...]
    z1_ref[...] = z1[:, None, :]
    a1 = jnp.where(z1 > 0.0, b1_ref[...], 0.0)
    a2 = _dot(_bf(a1), w2tb) + b2_ref[...]
    a2_ref[...] = a2[:, None, :]
    n1_ref[...] = jnp.sqrt(jnp.sum(w1t * w1t, axis=0, keepdims=True))


def _dist_kernel(xb_ref, w1tb_ref, w2tb_ref, w3tb_ref, z1_ref, a2_ref,
                 n1_ref, b3_ref, out_ref):
    bt = z1_ref.shape[0]
    d = w1tb_ref.shape[0]
    w1tb = w1tb_ref[...]
    bf0 = jnp.bfloat16(0)
    z1_rows = [z1_ref[i] for i in range(bt)]
    a1tb = jnp.concatenate(
        [jnp.where(z1_rows[i] > 0.0, w1tb, bf0) for i in range(bt)],
        axis=0)
    v2t_all = _dot(a1tb, w2tb_ref[...])
    v2tb_all = _bf(v2t_all)
    n2_rows, z2_rows, m2_rows = [], [], []
    for i in range(bt):
        v2t_i = v2t_all[i * d:(i + 1) * d]
        n2_rows.append(jnp.sqrt(jnp.sum(v2t_i * v2t_i, axis=0, keepdims=True)))
        z2 = _dot(xb_ref[i], v2tb_all[i * d:(i + 1) * d]) + a2_ref[i]
        z2_rows.append(z2)
        m2_rows.append(z2 > 0.0)
    v2mtb = jnp.concatenate(
        [jnp.where(m2_rows[i], v2tb_all[i * d:(i + 1) * d], bf0)
         for i in range(bt)], axis=0)
    v3t_all = _dot(v2mtb, w3tb_ref[...])
    v3tb_all = _bf(v3t_all)
    dist_rows = []
    for i in range(bt):
        v3t_i = v3t_all[i * d:(i + 1) * d]
        n3 = jnp.sqrt(jnp.sum(v3t_i * v3t_i, axis=0, keepdims=True))
        a2m = jnp.where(m2_rows[i], a2_ref[i], 0.0)
        a3 = _dot(_bf(a2m), w3tb_ref[...]) + b3_ref[...]
        z3 = _dot(xb_ref[i], v3tb_all[i * d:(i + 1) * d]) + a3
        d1 = jnp.abs(z1_rows[i]) / (n1_ref[...] + _EPS)
        d2 = jnp.abs(z2_rows[i]) / (n2_rows[i] + _EPS)
        d3 = jnp.abs(z3) / (n3 + _EPS)
        dist_rows.append(jnp.concatenate([d1, d2, d3], axis=1))
    out_ref[...] = jnp.concatenate(dist_rows, axis=0)[:, None, :]


def _sc_select(d_hbm, out_hbm, rows_v, stage_v):
    nc = 2
    wid = lax.axis_index("s") * nc + lax.axis_index("c")
    rows = rows_v.shape[0]
    n = rows_v.shape[1]
    pltpu.sync_copy(d_hbm.at[pl.ds(wid * rows, rows)], rows_v)
    lanes = lax.iota(jnp.int32, 16)
    inf16 = jnp.full((16,), jnp.inf, dtype=jnp.float32)
    zero16 = jnp.zeros((16,), dtype=jnp.float32)
    k16i = jnp.full((16,), _K, dtype=jnp.int32)

    def chunk_body(c, carry):
        new = []
        for r in range(rows):
            keep = list(carry[r])
            t = rows_v[r, pl.ds(c * 16, 16)]
            for j in range(_K):
                lo = jnp.minimum(keep[j], t)
                t = jnp.maximum(keep[j], t)
                keep[j] = lo
            new.append(tuple(keep))
        return tuple(new)

    init = tuple((inf16,) * _K for _ in range(rows))
    keeps = [list(ks) for ks in lax.fori_loop(0, n // 16, chunk_body, init)]

    need = [k16i] * rows
    total = [zero16] * rows
    for _ in range(_K):
        ms = []
        for r in range(rows):
            m = keeps[r][0]
            for j in range(1, _K):
                m = jnp.minimum(m, keeps[r][j])
            ms.append(m)
        for sh in (8, 4, 2, 1):
            for r in range(rows):
                stage_v[r] = ms[r]
            for r in range(rows):
                g = plsc.load_gather(
                    stage_v,
                    [jnp.full((16,), r, jnp.int32), (lanes + sh) & 15])
                ms[r] = jnp.minimum(ms[r], g)
        for r in range(rows):
            cnt = jnp.zeros((16,), dtype=jnp.int32)
            eqs = []
            for j in range(_K):
                eq = keeps[r][j] == ms[r]
                eqs.append(eq)
                cnt = cnt + plsc.all_reduce_population_count(eq)
            take = jnp.minimum(cnt, need[r])
            need[r] = need[r] - take
            contrib = ms[r] * take.astype(jnp.float32)
            total[r] = total[r] + jnp.where(take > 0, contrib, zero16)
            for j in range(_K):
                keeps[r][j] = jnp.where(eqs[j], inf16, keeps[r][j])
    acc = zero16
    for r in range(rows):
        acc = jnp.where(lanes == r, total[r], acc)
    stage_v[0] = acc
    pltpu.sync_copy(stage_v.at[0], out_hbm.at[wid])


def _select_kernel(d_ref, out_ref):
    v = d_ref[...].reshape(d_ref.shape[0], d_ref.shape[2])
    b, n = v.shape
    idx = jax.lax.broadcasted_iota(jnp.int32, (b, n), 1)
    acc = jnp.zeros((), dtype=jnp.float32)
    for _ in range(_K):
        row_min = jnp.min(v, axis=1, keepdims=True)
        acc = acc + jnp.sum(row_min)
        is_min = v == row_min
        min_idx = jnp.min(jnp.where(is_min, idx, n), axis=1, keepdims=True)
        v = jnp.where(idx == min_idx, jnp.float32(jnp.inf), v)
    out_ref[...] = jnp.broadcast_to(acc / _K, (1, 1))


@jax.jit
def kernel(inputs, W1, b1, W2, b2, W3, b3):
    B, d_in = inputs.shape
    h1 = W1.shape[0]
    h2 = W2.shape[0]
    h3 = W3.shape[0]
    b1r = b1[None, :]
    b2r = b2[None, :]
    b3r = b3[None, :]

    z1, a2, n1, w1tb, w2tb, w3tb, xb3 = pl.pallas_call(
        _fwd_kernel,
        out_shape=[
            jax.ShapeDtypeStruct((B, 1, h1), jnp.float32),
            jax.ShapeDtypeStruct((B, 1, h2), jnp.float32),
            jax.ShapeDtypeStruct((1, h1), jnp.float32),
            jax.ShapeDtypeStruct((d_in, h1), jnp.bfloat16),
            jax.ShapeDtypeStruct((h1, h2), jnp.bfloat16),
            jax.ShapeDtypeStruct((h2, h3), jnp.bfloat16),
            jax.ShapeDtypeStruct((B, 1, d_in), jnp.bfloat16),
        ],
    )(inputs, W1, W2, W3, b1r, b2r)

    n_total = h1 + h2 + h3
    bt = _BT
    n_workers = 32

    def dist_call(xb3_h, z1_h, a2_h):
        bh = xb3_h.shape[0]
        return pl.pallas_call(
            _dist_kernel,
            grid=(bh // bt,),
            in_specs=[
                pl.BlockSpec((bt, 1, d_in), lambda b: (b, 0, 0)),
                pl.BlockSpec((d_in, h1), lambda b: (0, 0)),
                pl.BlockSpec((h1, h2), lambda b: (0, 0)),
                pl.BlockSpec((h2, h3), lambda b: (0, 0)),
                pl.BlockSpec((bt, 1, h1), lambda b: (b, 0, 0)),
                pl.BlockSpec((bt, 1, h2), lambda b: (b, 0, 0)),
                pl.BlockSpec((1, h1), lambda b: (0, 0)),
                pl.BlockSpec((1, h3), lambda b: (0, 0)),
            ],
            out_specs=pl.BlockSpec((bt, 1, n_total), lambda b: (b, 0, 0)),
            out_shape=jax.ShapeDtypeStruct((bh, 1, n_total), jnp.float32),
        )(xb3_h, w1tb, w2tb, w3tb, z1_h, a2_h, n1, b3r)

    def select_call(dists_h):
        bh = dists_h.shape[0]
        sc_fn = functools.partial(
            pl.kernel,
            mesh=plsc.VectorSubcoreMesh(core_axis_name="c",
                                        subcore_axis_name="s"),
            out_type=jax.ShapeDtypeStruct((n_workers, 16), jnp.float32),
            compiler_params=pltpu.CompilerParams(needs_layout_passes=False),
            scratch_types=[
                pltpu.VMEM((bh // n_workers, n_total), jnp.float32),
                pltpu.VMEM((bh // n_workers, 16), jnp.float32),
            ],
        )(_sc_select)
        return sc_fn(dists_h.reshape(bh, n_total))

    dists = dist_call(xb3, z1, a2)
    part = select_call(dists)

    border_dist_sum = jnp.sum(part) / _K
    fct_dist_sum = jnp.zeros((), dtype=inputs.dtype)
    return (border_dist_sum, fct_dist_sum)


# speedup vs baseline: 1.2057x; 1.0922x over previous
import functools
import jax
import jax.numpy as jnp
from jax import lax
from jax.experimental import pallas as pl
from jax.experimental.pallas import tpu as pltpu
from jax.experimental.pallas import tpu_sc as plsc

_K = 8
_EPS = 1e-12
_BT = 16


def _bf(x):
    return x.astype(jnp.bfloat16)


def _dot(a, b):
    return jnp.dot(a, b, preferred_element_type=jnp.float32)


def _fwd_kernel(x_ref, w1_ref, w2_ref, w3_ref, b1_ref, b2_ref,
                z1_ref, a2_ref, n1_ref, w1tb_ref, w2tb_ref, w3tb_ref,
                xb3_ref):
    w1t = w1_ref[...].T
    w1tb = _bf(w1t)
    w1tb_ref[...] = w1tb
    w2tb = _bf(w2_ref[...].T)
    w2tb_ref[...] = w2tb
    w3tb_ref[...] = _bf(w3_ref[...].T)
    xb = _bf(x_ref[...])
    xb3_ref[...] = xb[:, None, :]
    z1 = _dot(xb, w1tb) + b1_ref[...]
    z1_ref[...] = z1[:, None, :]
    a1 = jnp.where(z1 > 0.0, b1_ref[...], 0.0)
    a2 = _dot(_bf(a1), w2tb) + b2_ref[...]
    a2_ref[...] = a2[:, None, :]
    n1_ref[...] = jnp.sqrt(jnp.sum(w1t * w1t, axis=0, keepdims=True))


def _dist_kernel(xb_ref, w1tb_ref, w2tb_ref, w3tb_ref, z1_ref, a2_ref,
                 n1_ref, b3_ref, out_ref):
    bt = z1_ref.shape[0]
    d = w1tb_ref.shape[0]
    w1tb = w1tb_ref[...]
    bf0 = jnp.bfloat16(0)
    z1_rows = [z1_ref[i] for i in range(bt)]
    a1tb = jnp.concatenate(
        [jnp.where(z1_rows[i] > 0.0, w1tb, bf0) for i in range(bt)],
        axis=0)
    v2t_all = _dot(a1tb, w2tb_ref[...])
    v2tb_all = _bf(v2t_all)
    n2sq_rows, z2_rows, m2_rows = [], [], []
    for i in range(bt):
        v2t_i = v2t_all[i * d:(i + 1) * d]
        n2sq_rows.append(jnp.sum(v2t_i * v2t_i, axis=0, keepdims=True))
        z2 = _dot(xb_ref[i], v2tb_all[i * d:(i + 1) * d]) + a2_ref[i]
        z2_rows.append(z2)
        m2_rows.append(z2 > 0.0)
    v2mtb = jnp.concatenate(
        [jnp.where(m2_rows[i], v2tb_all[i * d:(i + 1) * d], bf0)
         for i in range(bt)], axis=0)
    v3t_all = _dot(v2mtb, w3tb_ref[...])
    v3tb_all = _bf(v3t_all)
    a2m_all = jnp.concatenate(
        [jnp.where(m2_rows[i], a2_ref[i], 0.0) for i in range(bt)], axis=0)
    a3_all = _dot(_bf(a2m_all), w3tb_ref[...]) + b3_ref[...]
    n3sq_rows, z3_rows = [], []
    for i in range(bt):
        v3t_i = v3t_all[i * d:(i + 1) * d]
        n3sq_rows.append(jnp.sum(v3t_i * v3t_i, axis=0, keepdims=True))
        z3_rows.append(_dot(xb_ref[i], v3tb_all[i * d:(i + 1) * d]))
    z1_all = jnp.concatenate(z1_rows, axis=0)
    z2_all = jnp.concatenate(z2_rows, axis=0)
    z3_all = jnp.concatenate(z3_rows, axis=0) + a3_all
    n2_all = jnp.sqrt(jnp.concatenate(n2sq_rows, axis=0))
    n3_all = jnp.sqrt(jnp.concatenate(n3sq_rows, axis=0))
    d1 = jnp.abs(z1_all) / (n1_ref[...] + _EPS)
    d2 = jnp.abs(z2_all) / (n2_all + _EPS)
    d3 = jnp.abs(z3_all) / (n3_all + _EPS)
    out_ref[...] = jnp.concatenate([d1, d2, d3], axis=1)[:, None, :]


def _sc_select(d_hbm, out_hbm, rows_v, stage_v):
    nc = 2
    wid = lax.axis_index("s") * nc + lax.axis_index("c")
    rows = rows_v.shape[0]
    n = rows_v.shape[1]
    pltpu.sync_copy(d_hbm.at[pl.ds(wid * rows, rows)], rows_v)
    lanes = lax.iota(jnp.int32, 16)
    inf16 = jnp.full((16,), jnp.inf, dtype=jnp.float32)
    zero16 = jnp.zeros((16,), dtype=jnp.float32)
    k16i = jnp.full((16,), _K, dtype=jnp.int32)

    def chunk_body(c, carry):
        new = []
        for r in range(rows):
            keep = list(carry[r])
            t = rows_v[r, pl.ds(c * 16, 16)]
            for j in range(_K):
                lo = jnp.minimum(keep[j], t)
                t = jnp.maximum(keep[j], t)
                keep[j] = lo
            new.append(tuple(keep))
        return tuple(new)

    init = tuple((inf16,) * _K for _ in range(rows))
    keeps = [list(ks) for ks in lax.fori_loop(0, n // 16, chunk_body, init)]

    need = [k16i] * rows
    total = [zero16] * rows
    for _ in range(_K):
        ms = []
        for r in range(rows):
            m = keeps[r][0]
            for j in range(1, _K):
                m = jnp.minimum(m, keeps[r][j])
            ms.append(m)
        for sh in (8, 4, 2, 1):
            for r in range(rows):
                stage_v[r] = ms[r]
            for r in range(rows):
                g = plsc.load_gather(
                    stage_v,
                    [jnp.full((16,), r, jnp.int32), (lanes + sh) & 15])
                ms[r] = jnp.minimum(ms[r], g)
        for r in range(rows):
            cnt = jnp.zeros((16,), dtype=jnp.int32)
            eqs = []
            for j in range(_K):
                eq = keeps[r][j] == ms[r]
                eqs.append(eq)
                cnt = cnt + plsc.all_reduce_population_count(eq)
            take = jnp.minimum(cnt, need[r])
            need[r] = need[r] - take
            contrib = ms[r] * take.astype(jnp.float32)
            total[r] = total[r] + jnp.where(take > 0, contrib, zero16)
            for j in range(_K):
                keeps[r][j] = jnp.where(eqs[j], inf16, keeps[r][j])
    acc = zero16
    for r in range(rows):
        acc = jnp.where(lanes == r, total[r], acc)
    stage_v[0] = acc
    pltpu.sync_copy(stage_v.at[0], out_hbm.at[wid])


def _select_kernel(d_ref, out_ref):
    v = d_ref[...].reshape(d_ref.shape[0], d_ref.shape[2])
    b, n = v.shape
    idx = jax.lax.broadcasted_iota(jnp.int32, (b, n), 1)
    acc = jnp.zeros((), dtype=jnp.float32)
    for _ in range(_K):
        row_min = jnp.min(v, axis=1, keepdims=True)
        acc = acc + jnp.sum(row_min)
        is_min = v == row_min
        min_idx = jnp.min(jnp.where(is_min, idx, n), axis=1, keepdims=True)
        v = jnp.where(idx == min_idx, jnp.float32(jnp.inf), v)
    out_ref[...] = jnp.broadcast_to(acc / _K, (1, 1))


@jax.jit
def kernel(inputs, W1, b1, W2, b2, W3, b3):
    B, d_in = inputs.shape
    h1 = W1.shape[0]
    h2 = W2.shape[0]
    h3 = W3.shape[0]
    b1r = b1[None, :]
    b2r = b2[None, :]
    b3r = b3[None, :]

    z1, a2, n1, w1tb, w2tb, w3tb, xb3 = pl.pallas_call(
        _fwd_kernel,
        out_shape=[
            jax.ShapeDtypeStruct((B, 1, h1), jnp.float32),
            jax.ShapeDtypeStruct((B, 1, h2), jnp.float32),
            jax.ShapeDtypeStruct((1, h1), jnp.float32),
            jax.ShapeDtypeStruct((d_in, h1), jnp.bfloat16),
            jax.ShapeDtypeStruct((h1, h2), jnp.bfloat16),
            jax.ShapeDtypeStruct((h2, h3), jnp.bfloat16),
            jax.ShapeDtypeStruct((B, 1, d_in), jnp.bfloat16),
        ],
    )(inputs, W1, W2, W3, b1r, b2r)

    n_total = h1 + h2 + h3
    bt = _BT
    n_workers = 32

    def dist_call(xb3_h, z1_h, a2_h):
        bh = xb3_h.shape[0]
        return pl.pallas_call(
            _dist_kernel,
            grid=(bh // bt,),
            in_specs=[
                pl.BlockSpec((bt, 1, d_in), lambda b: (b, 0, 0)),
                pl.BlockSpec((d_in, h1), lambda b: (0, 0)),
                pl.BlockSpec((h1, h2), lambda b: (0, 0)),
                pl.BlockSpec((h2, h3), lambda b: (0, 0)),
                pl.BlockSpec((bt, 1, h1), lambda b: (b, 0, 0)),
                pl.BlockSpec((bt, 1, h2), lambda b: (b, 0, 0)),
                pl.BlockSpec((1, h1), lambda b: (0, 0)),
                pl.BlockSpec((1, h3), lambda b: (0, 0)),
            ],
            out_specs=pl.BlockSpec((bt, 1, n_total), lambda b: (b, 0, 0)),
            out_shape=jax.ShapeDtypeStruct((bh, 1, n_total), jnp.float32),
        )(xb3_h, w1tb, w2tb, w3tb, z1_h, a2_h, n1, b3r)

    def select_call(dists_h):
        bh = dists_h.shape[0]
        sc_fn = functools.partial(
            pl.kernel,
            mesh=plsc.VectorSubcoreMesh(core_axis_name="c",
                                        subcore_axis_name="s"),
            out_type=jax.ShapeDtypeStruct((n_workers, 16), jnp.float32),
            compiler_params=pltpu.CompilerParams(needs_layout_passes=False),
            scratch_types=[
                pltpu.VMEM((bh // n_workers, n_total), jnp.float32),
                pltpu.VMEM((bh // n_workers, 16), jnp.float32),
            ],
        )(_sc_select)
        return sc_fn(dists_h.reshape(bh, n_total))

    dists = dist_call(xb3, z1, a2)
    part = select_call(dists)

    border_dist_sum = jnp.sum(part) / _K
    fct_dist_sum = jnp.zeros((), dtype=inputs.dtype)
    return (border_dist_sum, fct_dist_sum)
